# baseline (speedup 1.0000x reference)
import functools

import jax
import jax.numpy as jnp
from jax import lax
from jax.experimental import pallas as pl
from jax.experimental.pallas import tpu as pltpu
from jax.experimental.pallas import tpu_sc as plsc

_B, _T, _IN_DIM = 16, 1024, 96
_CODE_DIM = 32
_K = 8192
_N = _B * _T
_TM = 256
_KC = 2048
_NBLK = _N // _TM

_SC_CORES = 2
_SC_SUBCORES = 16
_NW = _SC_CORES * _SC_SUBCORES
_BPW = _N // _NW
_CHUNK = 128
_NCH = _BPW // _CHUNK
_DPAD = 128


def _argmin_body(x_ref, we_ref, be_ref, cent_ref, out_ref):
    z = jnp.dot(x_ref[...], we_ref[...]) + be_ref[...]
    x_term = jnp.sum(z * z, axis=1, keepdims=True)
    best_val = None
    best_idx = None
    for j in range(_K // _KC):
        c = cent_ref[pl.ds(j * _KC, _KC), :]
        cross = -2.0 * lax.dot_general(z, c, (((1,), (1,)), ((), ())))
        y_term = jnp.sum(c * c, axis=1)[None, :]
        d = (cross + x_term) + y_term
        m = jnp.min(d, axis=1, keepdims=True)
        iota = lax.broadcasted_iota(jnp.int32, (_TM, _KC), 1) + j * _KC
        idx = jnp.min(jnp.where(d == m, iota, _K), axis=1, keepdims=True)
        if best_val is None:
            best_val, best_idx = m, idx
        else:
            if j * _KC == _K // 2:
                best_val = best_val.astype(jnp.bfloat16).astype(jnp.float32)
            take = m < best_val
            best_val = jnp.where(take, m, best_val)
            best_idx = jnp.where(take, idx, best_idx)
    out_ref[...] = best_idx


def _dtab_body(cent_ref, wd_ref, bd_ref, out_ref):
    out_ref[...] = jnp.dot(cent_ref[...], wd_ref[...]) + bd_ref[...]


def _sc_gather(dtab, idx2d):
    mesh = plsc.VectorSubcoreMesh(core_axis_name="c", subcore_axis_name="s")

    @functools.partial(
        pl.kernel,
        mesh=mesh,
        out_type=jax.ShapeDtypeStruct((_N, _DPAD), jnp.float32),
        scratch_types=[
            pltpu.VMEM((_NCH, _CHUNK), jnp.int32),
            pltpu.VMEM((_BPW, _DPAD), jnp.float32),
            pltpu.SemaphoreType.DMA,
        ],
    )
    def k(dtab_hbm, idx_hbm, out_hbm, idx_v, rows_v, sem):
        wid = lax.axis_index("s") * _SC_CORES + lax.axis_index("c")
        pltpu.sync_copy(idx_hbm.at[pl.ds(wid * _NCH, _NCH)], idx_v)
        copies = []
        for j in range(_NCH):
            copies.append(
                pltpu.async_copy(
                    dtab_hbm.at[idx_v.at[j]],
                    rows_v.at[pl.ds(j * _CHUNK, _CHUNK)],
                    sem,
                )
            )
        for c in copies:
            c.wait()
        pltpu.sync_copy(rows_v, out_hbm.at[pl.ds(wid * _BPW, _BPW)])

    return k(dtab, idx2d)


def kernel(x, W_enc, b_enc, centroids, W_dec, b_dec):
    flat_x = x.reshape(_N, _IN_DIM)
    be = b_enc.reshape(1, _CODE_DIM)
    bd = b_dec.reshape(1, _IN_DIM)

    align = pl.pallas_call(
        _argmin_body,
        grid=(_NBLK,),
        in_specs=[
            pl.BlockSpec((_TM, _IN_DIM), lambda i: (i, 0)),
            pl.BlockSpec((_IN_DIM, _CODE_DIM), lambda i: (0, 0)),
            pl.BlockSpec((1, _CODE_DIM), lambda i: (0, 0)),
            pl.BlockSpec((_K, _CODE_DIM), lambda i: (0, 0)),
        ],
        out_specs=pl.BlockSpec((_TM, 1), lambda i: (i, 0)),
        out_shape=jax.ShapeDtypeStruct((_N, 1), jnp.int32),
    )(flat_x, W_enc, be, centroids)

    wd_p = jnp.pad(W_dec, ((0, 0), (0, _DPAD - _IN_DIM)))
    bd_p = jnp.pad(bd, ((0, 0), (0, _DPAD - _IN_DIM)))
    dtab = pl.pallas_call(
        _dtab_body,
        in_specs=[
            pl.BlockSpec((_K, _CODE_DIM), lambda: (0, 0)),
            pl.BlockSpec((_CODE_DIM, _DPAD), lambda: (0, 0)),
            pl.BlockSpec((1, _DPAD), lambda: (0, 0)),
        ],
        out_specs=pl.BlockSpec((_K, _DPAD), lambda: (0, 0)),
        out_shape=jax.ShapeDtypeStruct((_K, _DPAD), jnp.float32),
    )(centroids, wd_p, bd_p)

    idx2d = align.reshape(_NW * _NCH, _CHUNK)
    out_flat = _sc_gather(dtab, idx2d)
    return out_flat[:, :_IN_DIM].reshape(_B, _T, _IN_DIM)


# speedup vs baseline: 1.2248x; 1.2248x over previous
import functools

import jax
import jax.numpy as jnp
from jax import lax
from jax.experimental import pallas as pl
from jax.experimental.pallas import tpu as pltpu
from jax.experimental.pallas import tpu_sc as plsc

_B, _T, _IN_DIM = 16, 1024, 96
_CODE_DIM = 32
_K = 8192
_N = _B * _T
_TM = 256
_KC = 2048
_NBLK = _N // _TM

_SC_CORES = 2
_SC_SUBCORES = 16
_NW = _SC_CORES * _SC_SUBCORES
_BPW = _N // _NW
_CHUNK = 128
_NCH = _BPW // _CHUNK
_DPAD = 128


def _argmin_body(x_ref, we_ref, be_ref, cent_ref, yterm_ref, out_ref):
    z = jnp.dot(x_ref[...], we_ref[...]) + be_ref[...]
    x_term = jnp.sum(z * z, axis=1, keepdims=True)
    z2 = z * -2.0
    iota = lax.broadcasted_iota(jnp.int32, (_TM, _KC), 1)
    best_val = None
    best_idx = None
    for j in range(_K // _KC):
        c = cent_ref[pl.ds(j * _KC, _KC), :]
        cross = lax.dot_general(z2, c, (((1,), (1,)), ((), ())))
        y_term = yterm_ref[:, pl.ds(j * _KC, _KC)]
        d = (cross + x_term) + y_term
        m = jnp.min(d, axis=1, keepdims=True)
        idx = jnp.min(jnp.where(d == m, iota, _KC),
                      axis=1, keepdims=True) + (j * _KC)
        if best_val is None:
            best_val, best_idx = m, idx
        else:
            if j * _KC == _K // 2:
                best_val = best_val.astype(jnp.bfloat16).astype(jnp.float32)
            take = m < best_val
            best_val = jnp.where(take, m, best_val)
            best_idx = jnp.where(take, idx, best_idx)
    out_ref[...] = best_idx


def _dtab_body(cent_ref, wd_ref, bd_ref, out_ref, yterm_ref):
    c = cent_ref[...]
    out_ref[...] = jnp.dot(c, wd_ref[...]) + bd_ref[...]
    yterm_ref[...] = jnp.sum(c * c, axis=1)[None, :]


def _sc_gather(dtab, idx2d):
    mesh = plsc.VectorSubcoreMesh(core_axis_name="c", subcore_axis_name="s")

    @functools.partial(
        pl.kernel,
        mesh=mesh,
        out_type=jax.ShapeDtypeStruct((_N, _DPAD), jnp.float32),
        scratch_types=[
            pltpu.VMEM((_NCH, _CHUNK), jnp.int32),
            pltpu.VMEM((_BPW, _DPAD), jnp.float32),
            pltpu.SemaphoreType.DMA,
        ],
    )
    def k(dtab_hbm, idx_hbm, out_hbm, idx_v, rows_v, sem):
        wid = lax.axis_index("s") * _SC_CORES + lax.axis_index("c")
        pltpu.sync_copy(idx_hbm.at[pl.ds(wid * _NCH, _NCH)], idx_v)
        copies = []
        for j in range(_NCH):
            copies.append(
                pltpu.async_copy(
                    dtab_hbm.at[idx_v.at[j]],
                    rows_v.at[pl.ds(j * _CHUNK, _CHUNK)],
                    sem,
                )
            )
        for c in copies:
            c.wait()
        pltpu.sync_copy(rows_v, out_hbm.at[pl.ds(wid * _BPW, _BPW)])

    return k(dtab, idx2d)


def kernel(x, W_enc, b_enc, centroids, W_dec, b_dec):
    flat_x = x.reshape(_N, _IN_DIM)
    be = b_enc.reshape(1, _CODE_DIM)
    bd = b_dec.reshape(1, _IN_DIM)

    wd_p = jnp.pad(W_dec, ((0, 0), (0, _DPAD - _IN_DIM)))
    bd_p = jnp.pad(bd, ((0, 0), (0, _DPAD - _IN_DIM)))
    dtab, yterm = pl.pallas_call(
        _dtab_body,
        in_specs=[
            pl.BlockSpec((_K, _CODE_DIM), lambda: (0, 0)),
            pl.BlockSpec((_CODE_DIM, _DPAD), lambda: (0, 0)),
            pl.BlockSpec((1, _DPAD), lambda: (0, 0)),
        ],
        out_specs=[
            pl.BlockSpec((_K, _DPAD), lambda: (0, 0)),
            pl.BlockSpec((1, _K), lambda: (0, 0)),
        ],
        out_shape=[
            jax.ShapeDtypeStruct((_K, _DPAD), jnp.float32),
            jax.ShapeDtypeStruct((1, _K), jnp.float32),
        ],
    )(centroids, wd_p, bd_p)

    align = pl.pallas_call(
        _argmin_body,
        grid=(_NBLK,),
        in_specs=[
            pl.BlockSpec((_TM, _IN_DIM), lambda i: (i, 0)),
            pl.BlockSpec((_IN_DIM, _CODE_DIM), lambda i: (0, 0)),
            pl.BlockSpec((1, _CODE_DIM), lambda i: (0, 0)),
            pl.BlockSpec((_K, _CODE_DIM), lambda i: (0, 0)),
            pl.BlockSpec((1, _K), lambda i: (0, 0)),
        ],
        out_specs=pl.BlockSpec((_TM, 1), lambda i: (i, 0)),
        out_shape=jax.ShapeDtypeStruct((_N, 1), jnp.int32),
    )(flat_x, W_enc, be, centroids, yterm)

    idx2d = align.reshape(_NW * _NCH, _CHUNK)
    out_flat = _sc_gather(dtab, idx2d)
    return out_flat[:, :_IN_DIM].reshape(_B, _T, _IN_DIM)


# speedup vs baseline: 1.3548x; 1.1061x over previous
import functools

import jax
import jax.numpy as jnp
from jax import lax
from jax.experimental import pallas as pl
from jax.experimental.pallas import tpu as pltpu
from jax.experimental.pallas import tpu_sc as plsc

_B, _T, _IN_DIM = 16, 1024, 96
_CODE_DIM = 32
_K = 8192
_N = _B * _T
_TM = 256
_KC = 2048
_NBLK = _N // _TM

_SC_CORES = 2
_SC_SUBCORES = 16
_NW = _SC_CORES * _SC_SUBCORES
_BPW = _N // _NW
_CHUNK = 128
_NCH = _BPW // _CHUNK
_DPAD = 128


def _argmin_body(x_ref, we_ref, be_ref, cent_ref, yterm_ref, iota_ref, out_ref):
    z = jnp.dot(x_ref[...], we_ref[...]) + be_ref[...]
    x_term = jnp.sum(z * z, axis=1, keepdims=True)
    z2 = z * -2.0
    iota = iota_ref[...]
    best_val = None
    best_idx = None
    for j in range(_K // _KC):
        c = cent_ref[pl.ds(j * _KC, _KC), :]
        cross = lax.dot_general(z2, c, (((1,), (1,)), ((), ())))
        y_term = yterm_ref[:, pl.ds(j * _KC, _KC)]
        d = (cross + x_term) + y_term
        m = jnp.min(d, axis=1, keepdims=True)
        idx = jnp.min(jnp.where(d == m, iota, jnp.float32(_KC)),
                      axis=1, keepdims=True) + jnp.float32(j * _KC)
        if best_val is None:
            best_val, best_idx = m, idx
        else:
            if j * _KC == _K // 2:
                best_val = best_val.astype(jnp.bfloat16).astype(jnp.float32)
            take = m < best_val
            best_val = jnp.where(take, m, best_val)
            best_idx = jnp.where(take, idx, best_idx)
    out_ref[...] = best_idx.astype(jnp.int32)


def _dtab_body(cent_ref, wd_ref, bd_ref, out_ref, yterm_ref):
    c = cent_ref[...]
    out_ref[...] = jnp.dot(c, wd_ref[...]) + bd_ref[...]
    yterm_ref[...] = jnp.sum(c * c, axis=1)[None, :]


def _sc_gather(dtab, idx2d):
    mesh = plsc.VectorSubcoreMesh(core_axis_name="c", subcore_axis_name="s")

    @functools.partial(
        pl.kernel,
        mesh=mesh,
        out_type=jax.ShapeDtypeStruct((_N, _DPAD), jnp.float32),
        scratch_types=[
            pltpu.VMEM((_NCH, _CHUNK), jnp.int32),
            pltpu.VMEM((_BPW, _DPAD), jnp.float32),
            pltpu.SemaphoreType.DMA,
        ],
    )
    def k(dtab_hbm, idx_hbm, out_hbm, idx_v, rows_v, sem):
        wid = lax.axis_index("s") * _SC_CORES + lax.axis_index("c")
        pltpu.sync_copy(idx_hbm.at[pl.ds(wid * _NCH, _NCH)], idx_v)
        copies = []
        for j in range(_NCH):
            copies.append(
                pltpu.async_copy(
                    dtab_hbm.at[idx_v.at[j]],
                    rows_v.at[pl.ds(j * _CHUNK, _CHUNK)],
                    sem,
                )
            )
        for c in copies:
            c.wait()
        pltpu.sync_copy(rows_v, out_hbm.at[pl.ds(wid * _BPW, _BPW)])

    return k(dtab, idx2d)


def kernel(x, W_enc, b_enc, centroids, W_dec, b_dec):
    flat_x = x.reshape(_N, _IN_DIM)
    be = b_enc.reshape(1, _CODE_DIM)
    bd = b_dec.reshape(1, _IN_DIM)

    wd_p = jnp.pad(W_dec, ((0, 0), (0, _DPAD - _IN_DIM)))
    bd_p = jnp.pad(bd, ((0, 0), (0, _DPAD - _IN_DIM)))
    dtab, yterm = pl.pallas_call(
        _dtab_body,
        in_specs=[
            pl.BlockSpec((_K, _CODE_DIM), lambda: (0, 0)),
            pl.BlockSpec((_CODE_DIM, _DPAD), lambda: (0, 0)),
            pl.BlockSpec((1, _DPAD), lambda: (0, 0)),
        ],
        out_specs=[
            pl.BlockSpec((_K, _DPAD), lambda: (0, 0)),
            pl.BlockSpec((1, _K), lambda: (0, 0)),
        ],
        out_shape=[
            jax.ShapeDtypeStruct((_K, _DPAD), jnp.float32),
            jax.ShapeDtypeStruct((1, _K), jnp.float32),
        ],
    )(centroids, wd_p, bd_p)

    align = pl.pallas_call(
        _argmin_body,
        grid=(_NBLK,),
        in_specs=[
            pl.BlockSpec((_TM, _IN_DIM), lambda i: (i, 0)),
            pl.BlockSpec((_IN_DIM, _CODE_DIM), lambda i: (0, 0)),
            pl.BlockSpec((1, _CODE_DIM), lambda i: (0, 0)),
            pl.BlockSpec((_K, _CODE_DIM), lambda i: (0, 0)),
            pl.BlockSpec((1, _K), lambda i: (0, 0)),
            pl.BlockSpec((1, _KC), lambda i: (0, 0)),
        ],
        out_specs=pl.BlockSpec((_TM, 1), lambda i: (i, 0)),
        out_shape=jax.ShapeDtypeStruct((_N, 1), jnp.int32),
    )(flat_x, W_enc, be, centroids, yterm,
      jnp.arange(_KC, dtype=jnp.float32).reshape(1, _KC))

    idx2d = align.reshape(_NW * _NCH, _CHUNK)
    out_flat = _sc_gather(dtab, idx2d)
    return out_flat[:, :_IN_DIM].reshape(_B, _T, _IN_DIM)


# speedup vs baseline: 1.5045x; 1.1106x over previous
import functools

import jax
import jax.numpy as jnp
from jax import lax
from jax.experimental import pallas as pl
from jax.experimental.pallas import tpu as pltpu
from jax.experimental.pallas import tpu_sc as plsc

_B, _T, _IN_DIM = 16, 1024, 96
_CODE_DIM = 32
_K = 8192
_N = _B * _T
_TM = 1024
_KC = 2048
_NBLK = _N // _TM

_SC_CORES = 2
_SC_SUBCORES = 16
_NW = _SC_CORES * _SC_SUBCORES
_BPW = _N // _NW
_CHUNK = 128
_NCH = _BPW // _CHUNK
_DPAD = 128


def _argmin_body(x_ref, we_ref, be_ref, cent_ref, yterm_ref, iota_ref, out_ref):
    z = jnp.dot(x_ref[...], we_ref[...]) + be_ref[...]
    x_term = jnp.sum(z * z, axis=1, keepdims=True)
    z2 = z * -2.0
    iota = iota_ref[...]
    best_val = None
    best_idx = None
    for j in range(_K // _KC):
        c = cent_ref[pl.ds(j * _KC, _KC), :]
        cross = lax.dot_general(z2, c, (((1,), (1,)), ((), ())))
        y_term = yterm_ref[:, pl.ds(j * _KC, _KC)]
        d = (cross + x_term) + y_term
        m = jnp.min(d, axis=1, keepdims=True)
        idx = jnp.min(jnp.where(d == m, iota, jnp.float32(_KC)),
                      axis=1, keepdims=True) + jnp.float32(j * _KC)
        if best_val is None:
            best_val, best_idx = m, idx
        else:
            if j * _KC == _K // 2:
                best_val = best_val.astype(jnp.bfloat16).astype(jnp.float32)
            take = m < best_val
            best_val = jnp.where(take, m, best_val)
            best_idx = jnp.where(take, idx, best_idx)
    out_ref[...] = best_idx.astype(jnp.int32).reshape(_TM // _CHUNK, _CHUNK)


def _dtab_body(cent_ref, wd_ref, bd_ref, out_ref, yterm_ref):
    c = cent_ref[...]
    out_ref[...] = jnp.dot(c, wd_ref[...]) + bd_ref[...]
    yterm_ref[...] = jnp.sum(c * c, axis=1)[None, :]


def _sc_gather(dtab, idx2d):
    mesh = plsc.VectorSubcoreMesh(core_axis_name="c", subcore_axis_name="s")

    @functools.partial(
        pl.kernel,
        mesh=mesh,
        out_type=jax.ShapeDtypeStruct((_N, _DPAD), jnp.float32),
        scratch_types=[
            pltpu.VMEM((_NCH, _CHUNK), jnp.int32),
            pltpu.VMEM((_BPW, _DPAD), jnp.float32),
            pltpu.SemaphoreType.DMA,
        ],
    )
    def k(dtab_hbm, idx_hbm, out_hbm, idx_v, rows_v, sem):
        wid = lax.axis_index("s") * _SC_CORES + lax.axis_index("c")
        pltpu.sync_copy(idx_hbm.at[pl.ds(wid * _NCH, _NCH)], idx_v)
        copies = []
        for j in range(_NCH):
            copies.append(
                pltpu.async_copy(
                    dtab_hbm.at[idx_v.at[j]],
                    rows_v.at[pl.ds(j * _CHUNK, _CHUNK)],
                    sem,
                )
            )
        for c in copies:
            c.wait()
        pltpu.sync_copy(rows_v, out_hbm.at[pl.ds(wid * _BPW, _BPW)])

    return k(dtab, idx2d)


def kernel(x, W_enc, b_enc, centroids, W_dec, b_dec):
    flat_x = x.reshape(_N, _IN_DIM)
    be = b_enc.reshape(1, _CODE_DIM)
    bd = b_dec.reshape(1, _IN_DIM)

    wd_p = jnp.pad(W_dec, ((0, 0), (0, _DPAD - _IN_DIM)))
    bd_p = jnp.pad(bd, ((0, 0), (0, _DPAD - _IN_DIM)))
    dtab, yterm = pl.pallas_call(
        _dtab_body,
        in_specs=[
            pl.BlockSpec((_K, _CODE_DIM), lambda: (0, 0)),
            pl.BlockSpec((_CODE_DIM, _DPAD), lambda: (0, 0)),
            pl.BlockSpec((1, _DPAD), lambda: (0, 0)),
        ],
        out_specs=[
            pl.BlockSpec((_K, _DPAD), lambda: (0, 0)),
            pl.BlockSpec((1, _K), lambda: (0, 0)),
        ],
        out_shape=[
            jax.ShapeDtypeStruct((_K, _DPAD), jnp.float32),
            jax.ShapeDtypeStruct((1, _K), jnp.float32),
        ],
    )(centroids, wd_p, bd_p)

    align = pl.pallas_call(
        _argmin_body,
        grid=(_NBLK,),
        in_specs=[
            pl.BlockSpec((_TM, _IN_DIM), lambda i: (i, 0)),
            pl.BlockSpec((_IN_DIM, _CODE_DIM), lambda i: (0, 0)),
            pl.BlockSpec((1, _CODE_DIM), lambda i: (0, 0)),
            pl.BlockSpec((_K, _CODE_DIM), lambda i: (0, 0)),
            pl.BlockSpec((1, _K), lambda i: (0, 0)),
            pl.BlockSpec((1, _KC), lambda i: (0, 0)),
        ],
        out_specs=pl.BlockSpec((_TM // _CHUNK, _CHUNK), lambda i: (i, 0)),
        out_shape=jax.ShapeDtypeStruct((_N // _CHUNK, _CHUNK), jnp.int32),
    )(flat_x, W_enc, be, centroids, yterm,
      jnp.arange(_KC, dtype=jnp.float32).reshape(1, _KC))

    out_flat = _sc_gather(dtab, align)
    return out_flat[:, :_IN_DIM].reshape(_B, _T, _IN_DIM)


# speedup vs baseline: 1.5285x; 1.0160x over previous
import functools

import jax
import jax.numpy as jnp
from jax import lax
from jax.experimental import pallas as pl
from jax.experimental.pallas import tpu as pltpu
from jax.experimental.pallas import tpu_sc as plsc

_B, _T, _IN_DIM = 16, 1024, 96
_CODE_DIM = 32
_K = 8192
_N = _B * _T
_TM = 2048
_KC = 4096
_NBLK = _N // _TM

_SC_CORES = 2
_SC_SUBCORES = 16
_NW = _SC_CORES * _SC_SUBCORES
_BPW = _N // _NW
_CHUNK = 128
_NCH = _BPW // _CHUNK
_DPAD = 128


def _argmin_body(x_ref, we_ref, be_ref, cent_ref, yterm_ref, iota_ref, out_ref):
    z = jnp.dot(x_ref[...], we_ref[...]) + be_ref[...]
    x_term = jnp.sum(z * z, axis=1, keepdims=True)
    z2 = z * -2.0
    iota = iota_ref[...]
    best_val = None
    best_idx = None
    for j in range(_K // _KC):
        c = cent_ref[pl.ds(j * _KC, _KC), :]
        cross = lax.dot_general(z2, c, (((1,), (1,)), ((), ())))
        y_term = yterm_ref[:, pl.ds(j * _KC, _KC)]
        d = (cross + x_term) + y_term
        m = jnp.min(d, axis=1, keepdims=True)
        idx = jnp.min(jnp.where(d == m, iota, jnp.float32(_KC)),
                      axis=1, keepdims=True) + jnp.float32(j * _KC)
        if best_val is None:
            best_val, best_idx = m, idx
        else:
            if j * _KC == _K // 2:
                best_val = best_val.astype(jnp.bfloat16).astype(jnp.float32)
            take = m < best_val
            best_val = jnp.where(take, m, best_val)
            best_idx = jnp.where(take, idx, best_idx)
    out_ref[...] = best_idx.astype(jnp.int32).reshape(_TM // _CHUNK, _CHUNK)


def _dtab_body(cent_ref, wd_ref, bd_ref, out_ref, yterm_ref):
    c = cent_ref[...]
    out_ref[...] = jnp.dot(c, wd_ref[...]) + bd_ref[...]
    yterm_ref[...] = jnp.sum(c * c, axis=1)[None, :]


def _sc_gather(dtab, idx2d):
    mesh = plsc.VectorSubcoreMesh(core_axis_name="c", subcore_axis_name="s")

    @functools.partial(
        pl.kernel,
        mesh=mesh,
        out_type=jax.ShapeDtypeStruct((_N, _DPAD), jnp.float32),
        scratch_types=[
            pltpu.VMEM((_NCH, _CHUNK), jnp.int32),
            pltpu.VMEM((_BPW, _DPAD), jnp.float32),
            pltpu.SemaphoreType.DMA,
        ],
    )
    def k(dtab_hbm, idx_hbm, out_hbm, idx_v, rows_v, sem):
        wid = lax.axis_index("s") * _SC_CORES + lax.axis_index("c")
        pltpu.sync_copy(idx_hbm.at[pl.ds(wid * _NCH, _NCH)], idx_v)
        copies = []
        for j in range(_NCH):
            copies.append(
                pltpu.async_copy(
                    dtab_hbm.at[idx_v.at[j]],
                    rows_v.at[pl.ds(j * _CHUNK, _CHUNK)],
                    sem,
                )
            )
        for c in copies:
            c.wait()
        pltpu.sync_copy(rows_v, out_hbm.at[pl.ds(wid * _BPW, _BPW)])

    return k(dtab, idx2d)


def kernel(x, W_enc, b_enc, centroids, W_dec, b_dec):
    flat_x = x.reshape(_N, _IN_DIM)
    be = b_enc.reshape(1, _CODE_DIM)
    bd = b_dec.reshape(1, _IN_DIM)

    wd_p = jnp.pad(W_dec, ((0, 0), (0, _DPAD - _IN_DIM)))
    bd_p = jnp.pad(bd, ((0, 0), (0, _DPAD - _IN_DIM)))
    dtab, yterm = pl.pallas_call(
        _dtab_body,
        in_specs=[
            pl.BlockSpec((_K, _CODE_DIM), lambda: (0, 0)),
            pl.BlockSpec((_CODE_DIM, _DPAD), lambda: (0, 0)),
            pl.BlockSpec((1, _DPAD), lambda: (0, 0)),
        ],
        out_specs=[
            pl.BlockSpec((_K, _DPAD), lambda: (0, 0)),
            pl.BlockSpec((1, _K), lambda: (0, 0)),
        ],
        out_shape=[
            jax.ShapeDtypeStruct((_K, _DPAD), jnp.float32),
            jax.ShapeDtypeStruct((1, _K), jnp.float32),
        ],
    )(centroids, wd_p, bd_p)

    align = pl.pallas_call(
        _argmin_body,
        grid=(_NBLK,),
        in_specs=[
            pl.BlockSpec((_TM, _IN_DIM), lambda i: (i, 0)),
            pl.BlockSpec((_IN_DIM, _CODE_DIM), lambda i: (0, 0)),
            pl.BlockSpec((1, _CODE_DIM), lambda i: (0, 0)),
            pl.BlockSpec((_K, _CODE_DIM), lambda i: (0, 0)),
            pl.BlockSpec((1, _K), lambda i: (0, 0)),
            pl.BlockSpec((1, _KC), lambda i: (0, 0)),
        ],
        out_specs=pl.BlockSpec((_TM // _CHUNK, _CHUNK), lambda i: (i, 0)),
        out_shape=jax.ShapeDtypeStruct((_N // _CHUNK, _CHUNK), jnp.int32),
    )(flat_x, W_enc, be, centroids, yterm,
      jnp.arange(_KC, dtype=jnp.float32).reshape(1, _KC))

    out_flat = _sc_gather(dtab, align)
    return out_flat[:, :_IN_DIM].reshape(_B, _T, _IN_DIM)
